# baseline (speedup 1.0000x reference)
import functools

import jax
import jax.numpy as jnp
from jax import lax
from jax.experimental import pallas as pl
from jax.experimental.pallas import tpu as pltpu
from jax.experimental.pallas import tpu_sc as plsc

BATCH = 16384
EMB = 64
LANES = 128
NW = 32
ROWS_PER_W = BATCH // LANES // NW


def _gather_body(user_hbm, item_hbm, utab_hbm, itab_hbm, uout_hbm, iout_hbm,
                 uidx_v, iidx_v, urows_v, irows_v, usem, isem):
  wid = lax.axis_index("s") * 2 + lax.axis_index("c")
  base = wid * ROWS_PER_W
  pltpu.sync_copy(user_hbm.at[pl.ds(base, ROWS_PER_W)], uidx_v)
  pltpu.sync_copy(item_hbm.at[pl.ds(base, ROWS_PER_W)], iidx_v)
  copies = []
  for j in range(ROWS_PER_W):
    copies.append(pltpu.async_copy(utab_hbm.at[uidx_v.at[j]], urows_v.at[j], usem))
    copies.append(pltpu.async_copy(itab_hbm.at[iidx_v.at[j]], irows_v.at[j], isem))
  for c in copies:
    c.wait()
  pltpu.sync_copy(urows_v, uout_hbm.at[pl.ds(base, ROWS_PER_W)])
  pltpu.sync_copy(irows_v, iout_hbm.at[pl.ds(base, ROWS_PER_W)])


def _sc_gather(user2d, item2d, user_table, item_table):
  mesh = plsc.VectorSubcoreMesh(core_axis_name="c", subcore_axis_name="s")
  k = pl.kernel(
      _gather_body,
      out_type=[
          jax.ShapeDtypeStruct((BATCH // LANES, LANES, EMB), jnp.float32),
          jax.ShapeDtypeStruct((BATCH // LANES, LANES, EMB), jnp.float32),
      ],
      mesh=mesh,
      scratch_types=[
          pltpu.VMEM((ROWS_PER_W, LANES), jnp.int32),
          pltpu.VMEM((ROWS_PER_W, LANES), jnp.int32),
          pltpu.VMEM((ROWS_PER_W, LANES, EMB), jnp.float32),
          pltpu.VMEM((ROWS_PER_W, LANES, EMB), jnp.float32),
          pltpu.SemaphoreType.DMA,
          pltpu.SemaphoreType.DMA,
      ],
      compiler_params=pltpu.CompilerParams(use_tc_tiling_on_sc=False),
  )
  return k(user2d, item2d, user_table, item_table)


def _mlp_body(ue_ref, ie_ref, w1_ref, b1_ref, w2_ref, b2_ref, w3_ref, b3_ref,
              out_ref):
  ue = ue_ref[...]
  ie = ie_ref[...]
  w1 = w1_ref[...]
  dn = (((1,), (1,)), ((), ()))
  h = lax.dot_general(ue, w1[:, :EMB], dn, preferred_element_type=jnp.float32)
  h += lax.dot_general(ie, w1[:, EMB:], dn, preferred_element_type=jnp.float32)
  h = jnp.maximum(h + b1_ref[...][None, :], 0.0)
  h2 = lax.dot_general(h, w2_ref[...], dn, preferred_element_type=jnp.float32)
  h2 = jnp.maximum(h2 + b2_ref[...][None, :], 0.0)
  y = jnp.sum(h2 * w3_ref[...], axis=1, keepdims=True)
  out_ref[...] = y + b3_ref[...][None, :]


def _mlp(ue, ie, W1, b1, W2, b2, W3, b3, block=2048):
  nblk = BATCH // block
  return pl.pallas_call(
      _mlp_body,
      grid=(nblk,),
      in_specs=[
          pl.BlockSpec((block, EMB), lambda i: (i, 0)),
          pl.BlockSpec((block, EMB), lambda i: (i, 0)),
          pl.BlockSpec(W1.shape, lambda i: (0, 0)),
          pl.BlockSpec(b1.shape, lambda i: (0,)),
          pl.BlockSpec(W2.shape, lambda i: (0, 0)),
          pl.BlockSpec(b2.shape, lambda i: (0,)),
          pl.BlockSpec(W3.shape, lambda i: (0, 0)),
          pl.BlockSpec(b3.shape, lambda i: (0,)),
      ],
      out_specs=pl.BlockSpec((block, 1), lambda i: (i, 0)),
      out_shape=jax.ShapeDtypeStruct((BATCH, 1), jnp.float32),
      compiler_params=pltpu.CompilerParams(
          dimension_semantics=("parallel",)),
  )(ue, ie, W1, b1, W2, b2, W3, b3)


@jax.jit
def kernel(user, item, user_table, item_table, W1, b1, W2, b2, W3, b3):
  user2d = user.astype(jnp.int32).reshape(BATCH // LANES, LANES)
  item2d = item.astype(jnp.int32).reshape(BATCH // LANES, LANES)
  ue3, ie3 = _sc_gather(user2d, item2d, user_table, item_table)
  ue = ue3.reshape(BATCH, EMB)
  ie = ie3.reshape(BATCH, EMB)
  y = _mlp(ue, ie, W1, b1, W2, b2, W3, b3)
  return y.reshape(BATCH)


# speedup vs baseline: 1.5662x; 1.5662x over previous
import functools

import jax
import jax.numpy as jnp
from jax import lax
from jax.experimental import pallas as pl
from jax.experimental.pallas import tpu as pltpu
from jax.experimental.pallas import tpu_sc as plsc

BATCH = 16384
EMB = 64
NW = 32
B_PER_W = BATCH // NW
NG = B_PER_W // 16


def _gather_one(idx_v, tab_h, rows_v, sem):
  for g in range(NG):
    vec = idx_v[pl.ds(g * 16, 16)]
    for l in range(16):
      idx = jax.lax.squeeze(jax.lax.slice(vec, (l,), (l + 1,)), (0,))
      pltpu.async_copy(tab_h.at[pl.ds(idx, 1)],
                       rows_v.at[pl.ds(g * 16 + l, 1)], sem)


def _drain(tab_h, rows_v, sem, n):
  def body(j, carry):
    pltpu.make_async_copy(tab_h.at[pl.ds(0, 1)],
                          rows_v.at[pl.ds(0, 1)], sem).wait()
    return carry
  lax.fori_loop(0, n, body, 0)


def _gather_body(uidx_h, iidx_h, utab_h, itab_h, uout_h, iout_h,
                 uidx_v, iidx_v, rows_v, usem, isem):
  wid = lax.axis_index("s") * 2 + lax.axis_index("c")
  base = wid * B_PER_W
  pltpu.sync_copy(uidx_h.at[pl.ds(base, B_PER_W)], uidx_v)
  pltpu.sync_copy(iidx_h.at[pl.ds(base, B_PER_W)], iidx_v)
  _gather_one(uidx_v, utab_h, rows_v, usem)
  _drain(utab_h, rows_v, usem, B_PER_W)
  pltpu.sync_copy(rows_v, uout_h.at[pl.ds(base, B_PER_W)])
  _gather_one(iidx_v, itab_h, rows_v, isem)
  _drain(itab_h, rows_v, isem, B_PER_W)
  pltpu.sync_copy(rows_v, iout_h.at[pl.ds(base, B_PER_W)])


def _sc_gather(user, item, user_table, item_table):
  mesh = plsc.VectorSubcoreMesh(core_axis_name="c", subcore_axis_name="s")
  k = pl.kernel(
      _gather_body,
      out_type=[
          jax.ShapeDtypeStruct((BATCH, EMB), jnp.float32),
          jax.ShapeDtypeStruct((BATCH, EMB), jnp.float32),
      ],
      mesh=mesh,
      scratch_types=[
          pltpu.VMEM((B_PER_W,), jnp.int32),
          pltpu.VMEM((B_PER_W,), jnp.int32),
          pltpu.VMEM((B_PER_W, EMB), jnp.float32),
          pltpu.SemaphoreType.DMA,
          pltpu.SemaphoreType.DMA,
      ],
      compiler_params=pltpu.CompilerParams(needs_layout_passes=False),
  )
  return k(user, item, user_table, item_table)


def _mlp_body(ue_ref, ie_ref, w1_ref, b1_ref, w2_ref, b2_ref, w3_ref, b3_ref,
              out_ref):
  ue = ue_ref[...]
  ie = ie_ref[...]
  w1 = w1_ref[...]
  dn = (((1,), (1,)), ((), ()))
  h = lax.dot_general(ue, w1[:, :EMB], dn, preferred_element_type=jnp.float32)
  h += lax.dot_general(ie, w1[:, EMB:], dn, preferred_element_type=jnp.float32)
  h = jnp.maximum(h + b1_ref[...][None, :], 0.0)
  h2 = lax.dot_general(h, w2_ref[...], dn, preferred_element_type=jnp.float32)
  h2 = jnp.maximum(h2 + b2_ref[...][None, :], 0.0)
  y = jnp.sum(h2 * w3_ref[...], axis=1, keepdims=True)
  out_ref[...] = y + b3_ref[...][None, :]


def _mlp(ue, ie, W1, b1, W2, b2, W3, b3, block=2048):
  nblk = BATCH // block
  return pl.pallas_call(
      _mlp_body,
      grid=(nblk,),
      in_specs=[
          pl.BlockSpec((block, EMB), lambda i: (i, 0)),
          pl.BlockSpec((block, EMB), lambda i: (i, 0)),
          pl.BlockSpec(W1.shape, lambda i: (0, 0)),
          pl.BlockSpec(b1.shape, lambda i: (0,)),
          pl.BlockSpec(W2.shape, lambda i: (0, 0)),
          pl.BlockSpec(b2.shape, lambda i: (0,)),
          pl.BlockSpec(W3.shape, lambda i: (0, 0)),
          pl.BlockSpec(b3.shape, lambda i: (0,)),
      ],
      out_specs=pl.BlockSpec((block, 1), lambda i: (i, 0)),
      out_shape=jax.ShapeDtypeStruct((BATCH, 1), jnp.float32),
      compiler_params=pltpu.CompilerParams(
          dimension_semantics=("parallel",)),
  )(ue, ie, W1, b1, W2, b2, W3, b3)


@jax.jit
def kernel(user, item, user_table, item_table, W1, b1, W2, b2, W3, b3):
  user = user.astype(jnp.int32)
  item = item.astype(jnp.int32)
  ue, ie = _sc_gather(user, item, user_table, item_table)
  y = _mlp(ue, ie, W1, b1, W2, b2, W3, b3)
  return y.reshape(BATCH)
